# baseline (speedup 1.0000x reference)
import functools
import jax
import jax.numpy as jnp
from jax.experimental import pallas as pl
from jax.experimental.pallas import tpu as pltpu

_S = 25
_NA = 8
_GS = (1, 4, 9, 25, 2, 3, 6, 25)


def _tc_body(ti_ref, p_ref, a_ref, out_ref):
    bt = p_ref.shape[0]
    f32 = jnp.float32

    ti = ti_ref[:, 0]
    idx = jnp.clip(ti - 8, 0, _NA - 1)
    A = a_ref[:]
    p = p_ref[:]

    neq = (A != p[:, None, :]).astype(f32)

    onehot = (jax.lax.broadcasted_iota(jnp.int32, (bt, _NA), 1)
              == idx[:, None]).astype(f32)

    grp = jnp.sum(A[:, :, 0] * onehot, axis=1)
    gi = jnp.clip(grp.astype(jnp.int32), 0, 7)
    cnt = jnp.full((bt,), _GS[-1], dtype=jnp.int32)
    for j in range(_NA - 2, -1, -1):
        cnt = jnp.where(gi == j, _GS[j], cnt)

    neq2 = neq.reshape(bt * _NA, 101)
    em = (jax.lax.broadcasted_iota(jnp.int32, (75, _S), 0) // 3
          == jax.lax.broadcasted_iota(jnp.int32, (75, _S), 1)).astype(f32)
    pne2 = jax.lax.dot_general(neq2[:, 26:101], em,
                               (((1,), (0,)), ((), ())),
                               preferred_element_type=f32)
    pne = pne2.reshape(bt, _NA, _S)

    slotm = A[:, :, 1:26] > 0.0
    k_iota = jax.lax.broadcasted_iota(jnp.int32, (bt, _NA, _S), 2)
    rangem = k_iota < cnt[:, None, None]

    d = jnp.sum(jnp.where(rangem, neq[:, :, 1:26], 0.0)
                + jnp.where(slotm, pne, 0.0), axis=2)
    r = jnp.sum(neq, axis=2)

    zero = (d == 0.0).astype(f32)
    n_zero = jnp.sum(zero, axis=1)
    d_idx = jnp.sum(d * onehot, axis=1)
    r_idx = jnp.sum(r * onehot, axis=1)
    slotcnt = jnp.sum(slotm.astype(f32), axis=2)
    sc_idx = jnp.sum(slotcnt * onehot, axis=1)
    fm = cnt.astype(f32) + 3.0 * sc_idx
    tz = jnp.sum(zero * onehot, axis=1)

    gmis = (grp != p[:, 0]).astype(f32)
    ham_sum = d_idx + gmis
    acc_same = (d_idx == 0.0).astype(f32)
    hams = ham_sum / jnp.maximum(fm, 1.0)
    once = tz * (n_zero == 1.0).astype(f32)

    sums = [jnp.sum(acc_same), jnp.sum(r_idx), jnp.sum(ham_sum),
            jnp.sum(fm), jnp.sum(hams), jnp.sum(tz), jnp.sum(once)]

    @pl.when(pl.program_id(0) == 0)
    def _():
        for j in range(8):
            out_ref[j] = 0.0

    for j, v in enumerate(sums):
        out_ref[j] = out_ref[j] + v


@jax.jit
def kernel(target_index, predict, contexts):
    B = predict.shape[0]
    bt = 512
    ti = target_index.astype(jnp.int32)

    sums = pl.pallas_call(
        _tc_body,
        grid=(B // bt,),
        in_specs=[
            pl.BlockSpec((bt, 1), lambda i: (i, 0)),
            pl.BlockSpec((bt, 101), lambda i: (i, 0)),
            pl.BlockSpec((bt, _NA, 101), lambda i: (i, 1, 0)),
        ],
        out_specs=pl.BlockSpec(memory_space=pltpu.SMEM),
        out_shape=jax.ShapeDtypeStruct((8,), jnp.float32),
        compiler_params=pltpu.CompilerParams(
            dimension_semantics=("arbitrary",),
        ),
    )(ti, predict, contexts)

    bf = jnp.float32(B)
    acc_same = sums[0] / bf
    hamf = sums[1] / bf
    accuracy = 1.0 - sums[2] / (sums[3] + bf)
    ham = sums[2] / bf
    hams = sums[4] / bf
    up = sums[5] / bf
    low = sums[6] / bf
    return jnp.stack([acc_same, hamf, accuracy, ham, hams, up, low]).astype(jnp.float32)
